# baseline (speedup 1.0000x reference)
import functools

import jax
import jax.numpy as jnp
import numpy as np
from jax import lax
from jax.experimental import pallas as pl
from jax.experimental.pallas import tpu as pltpu
from jax.experimental.pallas import tpu_sc as plsc

_B, _N, _C, _M = 2, 16384, 128, 128
_NS = 512
_D = _C + 3
_NW = 32
_BOXES_PER_W = _B * _M // _NW
_IDXCAP = 544
_CHUNKS = _N // 16
_GROUP = 8
_NGROUP = _CHUNKS // _GROUP
_MG = 8
_SENT = _N
_POOL_EXTRA_WIDTH = 1.0


def _sc_pool(trt, params):
    mesh = plsc.VectorSubcoreMesh(core_axis_name="c", subcore_axis_name="s")

    @functools.partial(
        pl.kernel,
        mesh=mesh,
        out_type=[
            jax.ShapeDtypeStruct((_B, _D, _M, _NS), jnp.float32),
            jax.ShapeDtypeStruct((_B * _M,), jnp.int32),
        ],
        scratch_types=[
            pltpu.VMEM((3 * _N,), jnp.float32),
            pltpu.VMEM((_IDXCAP,), jnp.int32),
            pltpu.VMEM((_NS,), jnp.int32),
            pltpu.VMEM((2, _MG * _NS), jnp.int32),
            pltpu.VMEM((2, _N + 16), jnp.float32),
            pltpu.VMEM((2, _MG, _NS), jnp.float32),
            pltpu.VMEM((16,), jnp.int32),
            pltpu.VMEM((16,), jnp.int32),
            pltpu.VMEM((_BOXES_PER_W * 8 * 16,), jnp.float32),
            pltpu.VMEM_SHARED((_M * _NS,), jnp.int32),
            pltpu.SemaphoreType.DMA,
            pltpu.SemaphoreType.DMA,
            pltpu.SemaphoreType.DMA,
            pltpu.SemaphoreType.DMA,
        ],
        compiler_params=pltpu.CompilerParams(needs_layout_passes=False,
                                             use_tc_tiling_on_sc=True),
    )
    def k(trt_hbm, par_hbm, out_hbm, flag_hbm,
          pts_v, idx_v, selbox_v, selg_v, col_v, outg_v, flag_v, off_ref,
          par_v, sel_sh, semg0, semg1, semw0, semw1):
        c = lax.axis_index("c")
        s = lax.axis_index("s")
        wid = c * 16 + s
        b = c
        box0 = wid * _BOXES_PER_W

        for plane in range(3):
            pltpu.sync_copy(trt_hbm.at[b, plane],
                            pts_v.at[pl.ds(plane * _N, _N)])
        pltpu.sync_copy(par_hbm.at[pl.ds(box0 * 128, _BOXES_PER_W * 128)],
                        par_v)

        iota = lax.broadcasted_iota(jnp.int32, (16,), 0)

        def box_body(bi, flags):
            pb = bi * 128
            cxv = par_v[pl.ds(pb, 16)]
            cyv = par_v[pl.ds(pb + 16, 16)]
            czv = par_v[pl.ds(pb + 32, 16)]
            dxv = par_v[pl.ds(pb + 48, 16)]
            dyv = par_v[pl.ds(pb + 64, 16)]
            dzv = par_v[pl.ds(pb + 80, 16)]
            cav = par_v[pl.ds(pb + 96, 16)]
            sav = par_v[pl.ds(pb + 112, 16)]

            off_ref[...] = jnp.zeros((16,), jnp.int32)

            def grp_body(g, carry):
                offv0 = off_ref[...]

                @pl.when(offv0[0] < _NS)
                def _scan_group():
                    offv = offv0
                    for u in range(_GROUP):
                        base = (g * _GROUP + u) * 16
                        xv = pts_v[pl.ds(base, 16)]
                        yv = pts_v[pl.ds(_N + base, 16)]
                        zv = pts_v[pl.ds(2 * _N + base, 16)]
                        sx = xv - cxv
                        sy = yv - cyv
                        sz = zv - czv
                        xr = sx * cav - sy * sav
                        yr = sx * sav + sy * cav
                        m = ((jnp.abs(sz) <= dzv)
                             & (jnp.abs(xr) <= dxv)
                             & (jnp.abs(yr) <= dyv))
                        cs = plsc.cumsum(m.astype(jnp.int32))
                        pos = offv + cs - 1
                        wm = m & (pos < _IDXCAP)
                        plsc.store_scatter(idx_v, [pos], base + iota, mask=wm)
                        offv = offv + plsc.all_reduce_population_count(m)
                    off_ref[...] = offv

                return carry

            lax.fori_loop(0, _NGROUP, grp_body, 0)

            cnt_v = off_ref[...]
            cnt = cnt_v[0]
            safe_v = jnp.maximum(cnt_v, 1)
            for j in range(_NS // 16):
                ar = j * 16 + iota
                selv = jnp.where(ar < cnt_v, ar, ar % safe_v)
                pidx = plsc.load_gather(idx_v, [selv])
                pidx = jnp.minimum(jnp.maximum(pidx, 0), _N - 1)
                pidx = jnp.where(cnt_v > 0, pidx, _SENT)
                selbox_v[pl.ds(j * 16, 16)] = pidx

            m_local = s * _BOXES_PER_W + bi
            pltpu.sync_copy(selbox_v, sel_sh.at[pl.ds(m_local * _NS, _NS)])

            empty = jnp.full((16,), (cnt == 0).astype(jnp.int32), jnp.int32)
            flags = jnp.where(iota == bi, empty, flags)
            return flags

        flags = lax.fori_loop(0, _BOXES_PER_W, box_body,
                              jnp.zeros((16,), jnp.int32))
        flag_v[...] = flags
        pltpu.sync_copy(flag_v.at[pl.ds(0, _BOXES_PER_W)],
                        flag_hbm.at[pl.ds(box0, _BOXES_PER_W)])

        plsc.subcore_barrier()

        zf = jnp.zeros((16,), jnp.float32)
        n_cols = jnp.where(s < _D - 8 * 16, 9, 8)

        def col_body(ci, carry):
            cidx = s + ci * 16
            pltpu.sync_copy(trt_hbm.at[b, cidx], col_v.at[0, pl.ds(0, _N)])
            col_v[0, pl.ds(_N, 16)] = zf

            semg = (semg0, semg1)
            semw = (semw0, semw1)
            nblk = _M // _MG
            gsel = [None, None]
            wout = [None, None]
            gsel[0] = pltpu.async_copy(
                sel_sh.at[pl.ds(0, _MG * _NS)], selg_v.at[0], semg[0])
            for mg in range(nblk):
                p = mg % 2
                gsel[p].wait()
                if mg + 1 < nblk:
                    gsel[1 - p] = pltpu.async_copy(
                        sel_sh.at[pl.ds((mg + 1) * _MG * _NS, _MG * _NS)],
                        selg_v.at[1 - p], semg[1 - p])
                if wout[p] is not None:
                    wout[p].wait()

                def gat_body(t, carry2):
                    for u in range(4):
                        o = (t * 4 + u) * 16
                        pv = selg_v[p, pl.ds(o, 16)]
                        vals = plsc.load_gather(col_v, [zero16_0, pv])
                        outg_v[p, o // _NS, pl.ds(o % _NS, 16)] = vals
                    return carry2
                lax.fori_loop(0, (_MG * _NS) // 64, gat_body, 0)

                wout[p] = pltpu.async_copy(
                    outg_v.at[p],
                    out_hbm.at[b, cidx, pl.ds(mg * _MG, _MG)],
                    semw[p])
            for p in range(2):
                if wout[p] is not None:
                    wout[p].wait()
            return carry

        zero16_0 = jnp.zeros((16,), jnp.int32)
        lax.fori_loop(0, n_cols, col_body, 0)

    return k(trt, params)


def kernel(points, point_features, boxes3d):
    B, N, _ = points.shape
    M = boxes3d.shape[1]

    eb = boxes3d.at[..., 3:6].add(_POOL_EXTRA_WIDTH)
    eb = eb.at[..., 2].add(-_POOL_EXTRA_WIDTH / 2.0)
    cx, cy, cz, dx, dy, dz, rz = [eb[..., i] for i in range(7)]
    czc = cz + dz / 2.0
    cosa = jnp.cos(-rz)
    sina = jnp.sin(-rz)
    params = jnp.stack([cx, cy, czc, dx / 2.0, dy / 2.0, dz / 2.0, cosa, sina],
                       axis=-1)
    params = jnp.broadcast_to(params.reshape(B * M, 8, 1),
                              (B * M, 8, 16)).astype(jnp.float32)
    params = params.reshape(B * M * 8 * 16)

    trt = jnp.concatenate(
        [jnp.transpose(points, (0, 2, 1)),
         jnp.transpose(point_features, (0, 2, 1))], axis=1)

    out, flags = _sc_pool(trt, params)
    out = jnp.transpose(out, (0, 2, 3, 1))
    return out, flags.reshape(B, M)


# speedup vs baseline: 1.1841x; 1.1841x over previous
import functools

import jax
import jax.numpy as jnp
import numpy as np
from jax import lax
from jax.experimental import pallas as pl
from jax.experimental.pallas import tpu as pltpu
from jax.experimental.pallas import tpu_sc as plsc

_B, _N, _C, _M = 2, 16384, 128, 128
_NS = 512
_D = _C + 3
_NW = 32
_BOXES_PER_W = _B * _M // _NW
_IDXCAP = 544
_CHUNKS = _N // 16
_GROUP = 8
_NGROUP = _CHUNKS // _GROUP
_MG = 8
_SENT = _N
_POOL_EXTRA_WIDTH = 1.0


def _sc_pool(trt, params):
    mesh = plsc.VectorSubcoreMesh(core_axis_name="c", subcore_axis_name="s")

    @functools.partial(
        pl.kernel,
        mesh=mesh,
        out_type=[
            jax.ShapeDtypeStruct((_B, _D, _M, _NS), jnp.float32),
            jax.ShapeDtypeStruct((_B * _M,), jnp.int32),
        ],
        scratch_types=[
            pltpu.VMEM((3 * _N,), jnp.float32),
            pltpu.VMEM((_IDXCAP,), jnp.int32),
            pltpu.VMEM((_NS,), jnp.int32),
            pltpu.VMEM((_M // 2 * _NS,), jnp.int32),
            pltpu.VMEM((_N + 16,), jnp.float32),
            pltpu.VMEM((2, _MG, _NS), jnp.float32),
            pltpu.VMEM((16,), jnp.int32),
            pltpu.VMEM((16,), jnp.int32),
            pltpu.VMEM((_BOXES_PER_W * 8 * 16,), jnp.float32),
            pltpu.VMEM_SHARED((_M * _NS,), jnp.int32),
            pltpu.SemaphoreType.DMA,
            pltpu.SemaphoreType.DMA,
            pltpu.SemaphoreType.DMA,
            pltpu.SemaphoreType.DMA,
        ],
        compiler_params=pltpu.CompilerParams(needs_layout_passes=False,
                                             use_tc_tiling_on_sc=True),
    )
    def k(trt_hbm, par_hbm, out_hbm, flag_hbm,
          pts_v, idx_v, selbox_v, sel_all, col_v, outg_v, flag_v, off_ref,
          par_v, sel_sh, semg0, semg1, semw0, semw1):
        c = lax.axis_index("c")
        s = lax.axis_index("s")
        wid = c * 16 + s
        b = c
        box0 = wid * _BOXES_PER_W

        for plane in range(3):
            pltpu.sync_copy(trt_hbm.at[b, plane],
                            pts_v.at[pl.ds(plane * _N, _N)])
        pltpu.sync_copy(par_hbm.at[pl.ds(box0 * 128, _BOXES_PER_W * 128)],
                        par_v)

        iota = lax.broadcasted_iota(jnp.int32, (16,), 0)

        def box_body(bi, flags):
            pb = bi * 128
            cxv = par_v[pl.ds(pb, 16)]
            cyv = par_v[pl.ds(pb + 16, 16)]
            czv = par_v[pl.ds(pb + 32, 16)]
            dxv = par_v[pl.ds(pb + 48, 16)]
            dyv = par_v[pl.ds(pb + 64, 16)]
            dzv = par_v[pl.ds(pb + 80, 16)]
            cav = par_v[pl.ds(pb + 96, 16)]
            sav = par_v[pl.ds(pb + 112, 16)]

            off_ref[...] = jnp.zeros((16,), jnp.int32)

            def grp_body(g, carry):
                offv0 = off_ref[...]

                @pl.when(offv0[0] < _NS)
                def _scan_group():
                    offv = offv0
                    for u in range(_GROUP):
                        base = (g * _GROUP + u) * 16
                        xv = pts_v[pl.ds(base, 16)]
                        yv = pts_v[pl.ds(_N + base, 16)]
                        zv = pts_v[pl.ds(2 * _N + base, 16)]
                        sx = xv - cxv
                        sy = yv - cyv
                        sz = zv - czv
                        xr = sx * cav - sy * sav
                        yr = sx * sav + sy * cav
                        m = ((jnp.abs(sz) <= dzv)
                             & (jnp.abs(xr) <= dxv)
                             & (jnp.abs(yr) <= dyv))
                        cs = plsc.cumsum(m.astype(jnp.int32))
                        pos = offv + cs - 1
                        wm = m & (pos < _IDXCAP)
                        plsc.store_scatter(idx_v, [pos], base + iota, mask=wm)
                        offv = offv + plsc.all_reduce_population_count(m)
                    off_ref[...] = offv

                return carry

            lax.fori_loop(0, _NGROUP, grp_body, 0)

            cnt_v = off_ref[...]
            cnt = cnt_v[0]
            safe_v = jnp.maximum(cnt_v, 1)
            for j in range(_NS // 16):
                ar = j * 16 + iota
                selv = jnp.where(ar < cnt_v, ar, ar % safe_v)
                pidx = plsc.load_gather(idx_v, [selv])
                pidx = jnp.minimum(jnp.maximum(pidx, 0), _N - 1)
                pidx = jnp.where(cnt_v > 0, pidx, _SENT)
                selbox_v[pl.ds(j * 16, 16)] = pidx

            m_local = s * _BOXES_PER_W + bi
            pltpu.sync_copy(selbox_v, sel_sh.at[pl.ds(m_local * _NS, _NS)])

            empty = jnp.full((16,), (cnt == 0).astype(jnp.int32), jnp.int32)
            flags = jnp.where(iota == bi, empty, flags)
            return flags

        flags = lax.fori_loop(0, _BOXES_PER_W, box_body,
                              jnp.zeros((16,), jnp.int32))
        flag_v[...] = flags
        pltpu.sync_copy(flag_v.at[pl.ds(0, _BOXES_PER_W)],
                        flag_hbm.at[pl.ds(box0, _BOXES_PER_W)])

        plsc.subcore_barrier()

        zf = jnp.zeros((16,), jnp.float32)
        n_cols = jnp.where(s < _D - 8 * 16, 9, 8)
        half_words = _M // 2 * _NS

        for half in range(2):
            pltpu.sync_copy(sel_sh.at[pl.ds(half * half_words, half_words)],
                            sel_all)

            def col_body(ci, carry):
                cidx = s + ci * 16
                pltpu.sync_copy(trt_hbm.at[b, cidx], col_v.at[pl.ds(0, _N)])
                col_v[pl.ds(_N, 16)] = zf

                semw = (semw0, semw1)
                wout = [None, None]
                for mg in range(_M // 2 // _MG):
                    p = mg % 2
                    if wout[p] is not None:
                        wout[p].wait()

                    def gat_body(t, carry2):
                        for u in range(4):
                            o = (t * 4 + u) * 16
                            pv = sel_all[pl.ds(mg * _MG * _NS + o, 16)]
                            vals = plsc.load_gather(col_v, [pv])
                            outg_v[p, o // _NS, pl.ds(o % _NS, 16)] = vals
                        return carry2
                    lax.fori_loop(0, (_MG * _NS) // 64, gat_body, 0)

                    wout[p] = pltpu.async_copy(
                        outg_v.at[p],
                        out_hbm.at[b, cidx,
                                   pl.ds(half * (_M // 2) + mg * _MG, _MG)],
                        semw[p])
                for p in range(2):
                    if wout[p] is not None:
                        wout[p].wait()
                return carry

            lax.fori_loop(0, n_cols, col_body, 0)

    return k(trt, params)


def kernel(points, point_features, boxes3d):
    B, N, _ = points.shape
    M = boxes3d.shape[1]

    eb = boxes3d.at[..., 3:6].add(_POOL_EXTRA_WIDTH)
    eb = eb.at[..., 2].add(-_POOL_EXTRA_WIDTH / 2.0)
    cx, cy, cz, dx, dy, dz, rz = [eb[..., i] for i in range(7)]
    czc = cz + dz / 2.0
    cosa = jnp.cos(-rz)
    sina = jnp.sin(-rz)
    params = jnp.stack([cx, cy, czc, dx / 2.0, dy / 2.0, dz / 2.0, cosa, sina],
                       axis=-1)
    params = jnp.broadcast_to(params.reshape(B * M, 8, 1),
                              (B * M, 8, 16)).astype(jnp.float32)
    params = params.reshape(B * M * 8 * 16)

    trt = jnp.concatenate(
        [jnp.transpose(points, (0, 2, 1)),
         jnp.transpose(point_features, (0, 2, 1))], axis=1)

    out, flags = _sc_pool(trt, params)
    out = jnp.transpose(out, (0, 2, 3, 1))
    return out, flags.reshape(B, M)


# speedup vs baseline: 2.3296x; 1.9674x over previous
import functools

import jax
import jax.numpy as jnp
import numpy as np
from jax import lax
from jax.experimental import pallas as pl
from jax.experimental.pallas import tpu as pltpu
from jax.experimental.pallas import tpu_sc as plsc

_B, _N, _C, _M = 2, 16384, 128, 128
_NS = 512
_D = _C + 3
_NW = 32
_BOXES_PER_W = _B * _M // _NW
_IDXCAP = 544
_CHUNKS = _N // 16
_GROUP = 8
_NGROUP = _CHUNKS // _GROUP
_MG = 8
_SENT = _N
_POOL_EXTRA_WIDTH = 1.0


def _sc_pool(trt, params):
    mesh = plsc.VectorSubcoreMesh(core_axis_name="c", subcore_axis_name="s")

    @functools.partial(
        pl.kernel,
        mesh=mesh,
        out_type=[
            jax.ShapeDtypeStruct((_B, _D, _M, _NS), jnp.float32),
            jax.ShapeDtypeStruct((_B * _M,), jnp.int32),
        ],
        scratch_types=[
            pltpu.VMEM((3 * _N,), jnp.float32),
            pltpu.VMEM((_IDXCAP,), jnp.int32),
            pltpu.VMEM((_NS,), jnp.int32),
            pltpu.VMEM((_M // 2 * _NS,), jnp.int32),
            pltpu.VMEM((_N + 16,), jnp.float32),
            pltpu.VMEM((2, _MG, _NS), jnp.float32),
            pltpu.VMEM((16,), jnp.int32),
            pltpu.VMEM((16,), jnp.int32),
            pltpu.VMEM((_BOXES_PER_W * 8 * 16,), jnp.float32),
            pltpu.VMEM_SHARED((_M * _NS,), jnp.int32),
            pltpu.SemaphoreType.DMA,
            pltpu.SemaphoreType.DMA,
            pltpu.SemaphoreType.DMA,
            pltpu.SemaphoreType.DMA,
        ],
        compiler_params=pltpu.CompilerParams(needs_layout_passes=False,
                                             use_tc_tiling_on_sc=True),
    )
    def k(trt_hbm, par_hbm, out_hbm, flag_hbm,
          pts_v, idx_v, selbox_v, sel_all, col_v, outg_v, flag_v, off_ref,
          par_v, sel_sh, semg0, semg1, semw0, semw1):
        c = lax.axis_index("c")
        s = lax.axis_index("s")
        wid = c * 16 + s
        b = c
        box0 = wid * _BOXES_PER_W

        for plane in range(3):
            pltpu.sync_copy(trt_hbm.at[b, plane],
                            pts_v.at[pl.ds(plane * _N, _N)])
        pltpu.sync_copy(par_hbm.at[pl.ds(box0 * 128, _BOXES_PER_W * 128)],
                        par_v)

        iota = lax.broadcasted_iota(jnp.int32, (16,), 0)

        def box_body(bi, flags):
            pb = bi * 128
            cxv = par_v[pl.ds(pb, 16)]
            cyv = par_v[pl.ds(pb + 16, 16)]
            czv = par_v[pl.ds(pb + 32, 16)]
            dxv = par_v[pl.ds(pb + 48, 16)]
            dyv = par_v[pl.ds(pb + 64, 16)]
            dzv = par_v[pl.ds(pb + 80, 16)]
            cav = par_v[pl.ds(pb + 96, 16)]
            sav = par_v[pl.ds(pb + 112, 16)]

            off_ref[...] = jnp.zeros((16,), jnp.int32)

            def grp_body(g, carry):
                offv0 = off_ref[...]

                @pl.when(offv0[0] < _NS)
                def _scan_group():
                    offv = offv0
                    for u in range(_GROUP):
                        base = (g * _GROUP + u) * 16
                        xv = pts_v[pl.ds(base, 16)]
                        yv = pts_v[pl.ds(_N + base, 16)]
                        zv = pts_v[pl.ds(2 * _N + base, 16)]
                        sx = xv - cxv
                        sy = yv - cyv
                        sz = zv - czv
                        xr = sx * cav - sy * sav
                        yr = sx * sav + sy * cav
                        m = ((jnp.abs(sz) <= dzv)
                             & (jnp.abs(xr) <= dxv)
                             & (jnp.abs(yr) <= dyv))
                        cs = plsc.cumsum(m.astype(jnp.int32))
                        pos = offv + cs - 1
                        wm = m & (pos < _IDXCAP)
                        plsc.store_scatter(idx_v, [pos], base + iota, mask=wm)
                        offv = offv + plsc.all_reduce_population_count(m)
                    off_ref[...] = offv

                return carry

            lax.fori_loop(0, _NGROUP, grp_body, 0)

            cnt_v = off_ref[...]
            cnt = cnt_v[0]
            safe_v = jnp.maximum(cnt_v, 1)
            for j in range(_NS // 16):
                ar = j * 16 + iota
                selv = jnp.where(ar < cnt_v, ar, ar % safe_v)
                pidx = plsc.load_gather(idx_v, [selv])
                pidx = jnp.minimum(jnp.maximum(pidx, 0), _N - 1)
                pidx = jnp.where(cnt_v > 0, pidx, _SENT)
                selbox_v[pl.ds(j * 16, 16)] = pidx

            m_local = s * _BOXES_PER_W + bi
            pltpu.sync_copy(selbox_v, sel_sh.at[pl.ds(m_local * _NS, _NS)])

            empty = jnp.full((16,), (cnt == 0).astype(jnp.int32), jnp.int32)
            flags = jnp.where(iota == bi, empty, flags)
            return flags

        flags = lax.fori_loop(0, _BOXES_PER_W, box_body,
                              jnp.zeros((16,), jnp.int32))
        flag_v[...] = flags
        pltpu.sync_copy(flag_v.at[pl.ds(0, _BOXES_PER_W)],
                        flag_hbm.at[pl.ds(box0, _BOXES_PER_W)])

        plsc.subcore_barrier()

        zf = jnp.zeros((16,), jnp.float32)
        n_cols = jnp.where(s < _D - 8 * 16, 9, 8)
        half_words = _M // 2 * _NS

        for half in range(2):
            pltpu.sync_copy(sel_sh.at[pl.ds(half * half_words, half_words)],
                            sel_all)

            def col_body(ci, carry):
                cidx = s + ci * 16
                pltpu.sync_copy(trt_hbm.at[b, cidx], col_v.at[pl.ds(0, _N)])
                col_v[pl.ds(_N, 16)] = zf

                semw = (semw0, semw1)
                wout = [None, None]
                for mg in range(_M // 2 // _MG):
                    p = mg % 2
                    if wout[p] is not None:
                        wout[p].wait()

                    def gat_body(t, carry2):
                        UNR = 8
                        os_ = [(t * UNR + u) * 16 for u in range(UNR)]
                        pvs = [sel_all[pl.ds(mg * _MG * _NS + o, 16)]
                               for o in os_]
                        vls = [plsc.load_gather(col_v, [pv]) for pv in pvs]
                        for o, v in zip(os_, vls):
                            outg_v[p, o // _NS, pl.ds(o % _NS, 16)] = v
                        return carry2
                    lax.fori_loop(0, (_MG * _NS) // (16 * 8), gat_body, 0)

                    wout[p] = pltpu.async_copy(
                        outg_v.at[p],
                        out_hbm.at[b, cidx,
                                   pl.ds(half * (_M // 2) + mg * _MG, _MG)],
                        semw[p])
                for p in range(2):
                    if wout[p] is not None:
                        wout[p].wait()
                return carry

            lax.fori_loop(0, n_cols, col_body, 0)

    return k(trt, params)


def kernel(points, point_features, boxes3d):
    B, N, _ = points.shape
    M = boxes3d.shape[1]

    eb = boxes3d.at[..., 3:6].add(_POOL_EXTRA_WIDTH)
    eb = eb.at[..., 2].add(-_POOL_EXTRA_WIDTH / 2.0)
    cx, cy, cz, dx, dy, dz, rz = [eb[..., i] for i in range(7)]
    czc = cz + dz / 2.0
    cosa = jnp.cos(-rz)
    sina = jnp.sin(-rz)
    params = jnp.stack([cx, cy, czc, dx / 2.0, dy / 2.0, dz / 2.0, cosa, sina],
                       axis=-1)
    params = jnp.broadcast_to(params.reshape(B * M, 8, 1),
                              (B * M, 8, 16)).astype(jnp.float32)
    params = params.reshape(B * M * 8 * 16)

    trt = jnp.concatenate(
        [jnp.transpose(points, (0, 2, 1)),
         jnp.transpose(point_features, (0, 2, 1))], axis=1)

    out, flags = _sc_pool(trt, params)
    out = jnp.transpose(out, (0, 2, 3, 1))
    return out, flags.reshape(B, M)


# speedup vs baseline: 2.3940x; 1.0277x over previous
import functools

import jax
import jax.numpy as jnp
import numpy as np
from jax import lax
from jax.experimental import pallas as pl
from jax.experimental.pallas import tpu as pltpu
from jax.experimental.pallas import tpu_sc as plsc

_B, _N, _C, _M = 2, 16384, 128, 128
_NS = 512
_D = _C + 3
_NW = 32
_BOXES_PER_W = _B * _M // _NW
_IDXCAP = 544
_CHUNKS = _N // 16
_GROUP = 8
_NGROUP = _CHUNKS // _GROUP
_MG = 8
_SENT = _N
_POOL_EXTRA_WIDTH = 1.0


def _sc_pool(trt, params):
    mesh = plsc.VectorSubcoreMesh(core_axis_name="c", subcore_axis_name="s")

    @functools.partial(
        pl.kernel,
        mesh=mesh,
        out_type=[
            jax.ShapeDtypeStruct((_B, _D, _M, _NS), jnp.float32),
            jax.ShapeDtypeStruct((_B * _M,), jnp.int32),
        ],
        scratch_types=[
            pltpu.VMEM((3 * _N,), jnp.float32),
            pltpu.VMEM((_IDXCAP,), jnp.int32),
            pltpu.VMEM((_NS,), jnp.int32),
            pltpu.VMEM((_M // 2 * _NS,), jnp.int32),
            pltpu.VMEM((_N + 16,), jnp.float32),
            pltpu.VMEM((2, _MG, _NS), jnp.float32),
            pltpu.VMEM((16,), jnp.int32),
            pltpu.VMEM((16,), jnp.int32),
            pltpu.VMEM((_BOXES_PER_W * 8 * 16,), jnp.float32),
            pltpu.VMEM_SHARED((_M * _NS,), jnp.int32),
            pltpu.SemaphoreType.DMA,
            pltpu.SemaphoreType.DMA,
            pltpu.SemaphoreType.DMA,
            pltpu.SemaphoreType.DMA,
        ],
        compiler_params=pltpu.CompilerParams(needs_layout_passes=False,
                                             use_tc_tiling_on_sc=True),
    )
    def k(trt_hbm, par_hbm, out_hbm, flag_hbm,
          pts_v, idx_v, selbox_v, sel_all, col_v, outg_v, flag_v, off_ref,
          par_v, sel_sh, semg0, semg1, semw0, semw1):
        c = lax.axis_index("c")
        s = lax.axis_index("s")
        wid = c * 16 + s
        b = c
        box0 = wid * _BOXES_PER_W

        for plane in range(3):
            pltpu.sync_copy(trt_hbm.at[b, plane],
                            pts_v.at[pl.ds(plane * _N, _N)])
        pltpu.sync_copy(par_hbm.at[pl.ds(box0 * 128, _BOXES_PER_W * 128)],
                        par_v)

        iota = lax.broadcasted_iota(jnp.int32, (16,), 0)

        def box_body(bi, flags):
            pb = bi * 128
            cxv = par_v[pl.ds(pb, 16)]
            cyv = par_v[pl.ds(pb + 16, 16)]
            czv = par_v[pl.ds(pb + 32, 16)]
            dxv = par_v[pl.ds(pb + 48, 16)]
            dyv = par_v[pl.ds(pb + 64, 16)]
            dzv = par_v[pl.ds(pb + 80, 16)]
            cav = par_v[pl.ds(pb + 96, 16)]
            sav = par_v[pl.ds(pb + 112, 16)]

            off_ref[...] = jnp.zeros((16,), jnp.int32)

            def grp_body(g, carry):
                offv0 = off_ref[...]

                @pl.when(offv0[0] < _NS)
                def _scan_group():
                    offv = offv0
                    for u in range(_GROUP):
                        base = (g * _GROUP + u) * 16
                        xv = pts_v[pl.ds(base, 16)]
                        yv = pts_v[pl.ds(_N + base, 16)]
                        zv = pts_v[pl.ds(2 * _N + base, 16)]
                        sx = xv - cxv
                        sy = yv - cyv
                        sz = zv - czv
                        xr = sx * cav - sy * sav
                        yr = sx * sav + sy * cav
                        m = ((jnp.abs(sz) <= dzv)
                             & (jnp.abs(xr) <= dxv)
                             & (jnp.abs(yr) <= dyv))
                        cs = plsc.cumsum(m.astype(jnp.int32))
                        pos = offv + cs - 1
                        wm = m & (pos < _IDXCAP)
                        plsc.store_scatter(idx_v, [pos], base + iota, mask=wm)
                        offv = offv + plsc.all_reduce_population_count(m)
                    off_ref[...] = offv

                return carry

            lax.fori_loop(0, _NGROUP, grp_body, 0)

            cnt_v = off_ref[...]
            cnt = cnt_v[0]

            @pl.when(cnt >= _NS)
            def _sel_direct():
                for j in range(_NS // 16):
                    selbox_v[pl.ds(j * 16, 16)] = idx_v[pl.ds(j * 16, 16)]

            @pl.when(cnt < _NS)
            def _sel_wrapped():
                safe_v = jnp.maximum(cnt_v, 1)
                for j in range(_NS // 16):
                    ar = j * 16 + iota
                    selv = jnp.where(ar < cnt_v, ar, ar % safe_v)
                    pidx = plsc.load_gather(idx_v, [selv])
                    pidx = jnp.minimum(jnp.maximum(pidx, 0), _N - 1)
                    pidx = jnp.where(cnt_v > 0, pidx, _SENT)
                    selbox_v[pl.ds(j * 16, 16)] = pidx

            m_local = s * _BOXES_PER_W + bi
            pltpu.sync_copy(selbox_v, sel_sh.at[pl.ds(m_local * _NS, _NS)])

            empty = jnp.full((16,), (cnt == 0).astype(jnp.int32), jnp.int32)
            flags = jnp.where(iota == bi, empty, flags)
            return flags

        flags = lax.fori_loop(0, _BOXES_PER_W, box_body,
                              jnp.zeros((16,), jnp.int32))
        flag_v[...] = flags
        pltpu.sync_copy(flag_v.at[pl.ds(0, _BOXES_PER_W)],
                        flag_hbm.at[pl.ds(box0, _BOXES_PER_W)])

        plsc.subcore_barrier()

        zf = jnp.zeros((16,), jnp.float32)
        n_cols = jnp.where(s < _D - 8 * 16, 9, 8)
        half_words = _M // 2 * _NS

        for half in range(2):
            pltpu.sync_copy(sel_sh.at[pl.ds(half * half_words, half_words)],
                            sel_all)

            def col_body(ci, carry):
                cidx = s + ci * 16
                pltpu.sync_copy(trt_hbm.at[b, cidx], col_v.at[pl.ds(0, _N)])
                col_v[pl.ds(_N, 16)] = zf

                semw = (semw0, semw1)
                wout = [None, None]
                for mg in range(_M // 2 // _MG):
                    p = mg % 2
                    if wout[p] is not None:
                        wout[p].wait()

                    def gat_body(t, carry2):
                        UNR = 8
                        os_ = [(t * UNR + u) * 16 for u in range(UNR)]
                        pvs = [sel_all[pl.ds(mg * _MG * _NS + o, 16)]
                               for o in os_]
                        vls = [plsc.load_gather(col_v, [pv]) for pv in pvs]
                        for o, v in zip(os_, vls):
                            outg_v[p, o // _NS, pl.ds(o % _NS, 16)] = v
                        return carry2
                    lax.fori_loop(0, (_MG * _NS) // (16 * 8), gat_body, 0)

                    wout[p] = pltpu.async_copy(
                        outg_v.at[p],
                        out_hbm.at[b, cidx,
                                   pl.ds(half * (_M // 2) + mg * _MG, _MG)],
                        semw[p])
                for p in range(2):
                    if wout[p] is not None:
                        wout[p].wait()
                return carry

            lax.fori_loop(0, n_cols, col_body, 0)

    return k(trt, params)


def kernel(points, point_features, boxes3d):
    B, N, _ = points.shape
    M = boxes3d.shape[1]

    eb = boxes3d.at[..., 3:6].add(_POOL_EXTRA_WIDTH)
    eb = eb.at[..., 2].add(-_POOL_EXTRA_WIDTH / 2.0)
    cx, cy, cz, dx, dy, dz, rz = [eb[..., i] for i in range(7)]
    czc = cz + dz / 2.0
    cosa = jnp.cos(-rz)
    sina = jnp.sin(-rz)
    params = jnp.stack([cx, cy, czc, dx / 2.0, dy / 2.0, dz / 2.0, cosa, sina],
                       axis=-1)
    params = jnp.broadcast_to(params.reshape(B * M, 8, 1),
                              (B * M, 8, 16)).astype(jnp.float32)
    params = params.reshape(B * M * 8 * 16)

    trt = jnp.concatenate(
        [jnp.transpose(points, (0, 2, 1)),
         jnp.transpose(point_features, (0, 2, 1))], axis=1)

    out, flags = _sc_pool(trt, params)
    out = jnp.transpose(out, (0, 2, 3, 1))
    return out, flags.reshape(B, M)


# speedup vs baseline: 3.0448x; 1.2718x over previous
import functools

import jax
import jax.numpy as jnp
import numpy as np
from jax import lax
from jax.experimental import pallas as pl
from jax.experimental.pallas import tpu as pltpu
from jax.experimental.pallas import tpu_sc as plsc

_B, _N, _C, _M = 2, 16384, 128, 128
_NS = 512
_D = _C + 3
_NW = 32
_BOXES_PER_W = _B * _M // _NW
_IDXCAP = 544
_CHUNKS = _N // 16
_GROUP = 8
_NGROUP = _CHUNKS // _GROUP
_MG = 8
_SENT = _N
_POOL_EXTRA_WIDTH = 1.0


def _sc_pool(trt, params):
    mesh = plsc.VectorSubcoreMesh(core_axis_name="c", subcore_axis_name="s")

    @functools.partial(
        pl.kernel,
        mesh=mesh,
        out_type=[
            jax.ShapeDtypeStruct((_B, _D, _M, _NS), jnp.float32),
            jax.ShapeDtypeStruct((_B * _M,), jnp.int32),
        ],
        scratch_types=[
            pltpu.VMEM((3 * _N,), jnp.float32),
            pltpu.VMEM((_IDXCAP,), jnp.int32),
            pltpu.VMEM((_NS,), jnp.int32),
            pltpu.VMEM((_M // 2 * _NS,), jnp.int32),
            pltpu.VMEM((_N + 16,), jnp.float32),
            pltpu.VMEM((2, _MG, _NS), jnp.float32),
            pltpu.VMEM((16,), jnp.int32),
            pltpu.VMEM((16,), jnp.int32),
            pltpu.VMEM((_BOXES_PER_W * 8 * 16,), jnp.float32),
            pltpu.VMEM_SHARED((_M * _NS,), jnp.int32),
            pltpu.SemaphoreType.DMA,
            pltpu.SemaphoreType.DMA,
            pltpu.SemaphoreType.DMA,
            pltpu.SemaphoreType.DMA,
        ],
        compiler_params=pltpu.CompilerParams(needs_layout_passes=False,
                                             use_tc_tiling_on_sc=True),
    )
    def k(trt_hbm, par_hbm, out_hbm, flag_hbm,
          pts_v, idx_v, selbox_v, sel_all, col_v, outg_v, flag_v, off_ref,
          par_v, sel_sh, semg0, semg1, semw0, semw1):
        c = lax.axis_index("c")
        s = lax.axis_index("s")
        wid = c * 16 + s
        b = c
        box0 = wid * _BOXES_PER_W

        for plane in range(3):
            pltpu.sync_copy(trt_hbm.at[b, plane],
                            pts_v.at[pl.ds(plane * _N, _N)])
        pltpu.sync_copy(par_hbm.at[pl.ds(box0 * 128, _BOXES_PER_W * 128)],
                        par_v)

        iota = lax.broadcasted_iota(jnp.int32, (16,), 0)

        def box_body(bi, flags):
            pb = bi * 128
            cxv = par_v[pl.ds(pb, 16)]
            cyv = par_v[pl.ds(pb + 16, 16)]
            czv = par_v[pl.ds(pb + 32, 16)]
            dxv = par_v[pl.ds(pb + 48, 16)]
            dyv = par_v[pl.ds(pb + 64, 16)]
            dzv = par_v[pl.ds(pb + 80, 16)]
            cav = par_v[pl.ds(pb + 96, 16)]
            sav = par_v[pl.ds(pb + 112, 16)]

            off_ref[...] = jnp.zeros((16,), jnp.int32)

            def grp_body(g, carry):
                offv0 = off_ref[...]

                @pl.when(offv0[0] < _NS)
                def _scan_group():
                    bases = [(g * _GROUP + u) * 16 for u in range(_GROUP)]
                    xs = [pts_v[pl.ds(bb, 16)] for bb in bases]
                    ys = [pts_v[pl.ds(_N + bb, 16)] for bb in bases]
                    zs = [pts_v[pl.ds(2 * _N + bb, 16)] for bb in bases]
                    ms = []
                    for u in range(_GROUP):
                        sx = xs[u] - cxv
                        sy = ys[u] - cyv
                        sz = zs[u] - czv
                        xr = sx * cav - sy * sav
                        yr = sx * sav + sy * cav
                        ms.append((jnp.abs(sz) <= dzv)
                                  & (jnp.abs(xr) <= dxv)
                                  & (jnp.abs(yr) <= dyv))
                    css = [plsc.cumsum(m.astype(jnp.int32)) for m in ms]
                    pcs = [plsc.all_reduce_population_count(m) for m in ms]
                    offv = offv0
                    for u in range(_GROUP):
                        pos = offv + css[u] - 1
                        wm = ms[u] & (pos < _IDXCAP)
                        plsc.store_scatter(idx_v, [pos], bases[u] + iota,
                                           mask=wm)
                        offv = offv + pcs[u]
                    off_ref[...] = offv

                return carry

            lax.fori_loop(0, _NGROUP, grp_body, 0)

            cnt_v = off_ref[...]
            cnt = cnt_v[0]

            @pl.when(cnt >= _NS)
            def _sel_direct():
                for j in range(_NS // 16):
                    selbox_v[pl.ds(j * 16, 16)] = idx_v[pl.ds(j * 16, 16)]

            @pl.when(cnt < _NS)
            def _sel_wrapped():
                safe_v = jnp.maximum(cnt_v, 1)
                for j in range(_NS // 16):
                    ar = j * 16 + iota
                    selv = jnp.where(ar < cnt_v, ar, ar % safe_v)
                    pidx = plsc.load_gather(idx_v, [selv])
                    pidx = jnp.minimum(jnp.maximum(pidx, 0), _N - 1)
                    pidx = jnp.where(cnt_v > 0, pidx, _SENT)
                    selbox_v[pl.ds(j * 16, 16)] = pidx

            m_local = s * _BOXES_PER_W + bi
            pltpu.sync_copy(selbox_v, sel_sh.at[pl.ds(m_local * _NS, _NS)])

            empty = jnp.full((16,), (cnt == 0).astype(jnp.int32), jnp.int32)
            flags = jnp.where(iota == bi, empty, flags)
            return flags

        flags = lax.fori_loop(0, _BOXES_PER_W, box_body,
                              jnp.zeros((16,), jnp.int32))
        flag_v[...] = flags
        pltpu.sync_copy(flag_v.at[pl.ds(0, _BOXES_PER_W)],
                        flag_hbm.at[pl.ds(box0, _BOXES_PER_W)])

        plsc.subcore_barrier()

        zf = jnp.zeros((16,), jnp.float32)
        n_cols = jnp.where(s < _D - 8 * 16, 9, 8)
        half_words = _M // 2 * _NS

        for half in range(2):
            pltpu.sync_copy(sel_sh.at[pl.ds(half * half_words, half_words)],
                            sel_all)

            def col_body(ci, carry):
                cidx = s + ci * 16
                pltpu.sync_copy(trt_hbm.at[b, cidx], col_v.at[pl.ds(0, _N)])
                col_v[pl.ds(_N, 16)] = zf

                semw = (semw0, semw1)
                wout = [None, None]
                for mg in range(_M // 2 // _MG):
                    p = mg % 2
                    if wout[p] is not None:
                        wout[p].wait()

                    def gat_body(t, carry2):
                        UNR = 8
                        os_ = [(t * UNR + u) * 16 for u in range(UNR)]
                        pvs = [sel_all[pl.ds(mg * _MG * _NS + o, 16)]
                               for o in os_]
                        vls = [plsc.load_gather(col_v, [pv]) for pv in pvs]
                        for o, v in zip(os_, vls):
                            outg_v[p, o // _NS, pl.ds(o % _NS, 16)] = v
                        return carry2
                    lax.fori_loop(0, (_MG * _NS) // (16 * 8), gat_body, 0)

                    wout[p] = pltpu.async_copy(
                        outg_v.at[p],
                        out_hbm.at[b, cidx,
                                   pl.ds(half * (_M // 2) + mg * _MG, _MG)],
                        semw[p])
                for p in range(2):
                    if wout[p] is not None:
                        wout[p].wait()
                return carry

            lax.fori_loop(0, n_cols, col_body, 0)

    return k(trt, params)


def kernel(points, point_features, boxes3d):
    B, N, _ = points.shape
    M = boxes3d.shape[1]

    eb = boxes3d.at[..., 3:6].add(_POOL_EXTRA_WIDTH)
    eb = eb.at[..., 2].add(-_POOL_EXTRA_WIDTH / 2.0)
    cx, cy, cz, dx, dy, dz, rz = [eb[..., i] for i in range(7)]
    czc = cz + dz / 2.0
    cosa = jnp.cos(-rz)
    sina = jnp.sin(-rz)
    params = jnp.stack([cx, cy, czc, dx / 2.0, dy / 2.0, dz / 2.0, cosa, sina],
                       axis=-1)
    params = jnp.broadcast_to(params.reshape(B * M, 8, 1),
                              (B * M, 8, 16)).astype(jnp.float32)
    params = params.reshape(B * M * 8 * 16)

    trt = jnp.concatenate(
        [jnp.transpose(points, (0, 2, 1)),
         jnp.transpose(point_features, (0, 2, 1))], axis=1)

    out, flags = _sc_pool(trt, params)
    out = jnp.transpose(out, (0, 2, 3, 1))
    return out, flags.reshape(B, M)
